# Initial kernel scaffold; baseline (speedup 1.0000x reference)
import jax
import jax.numpy as jnp
from jax.experimental import pallas as pl


def kernel(img_batch, canvases):
    raise NotImplementedError("write your pallas kernel here")


# speedup vs baseline: 6.2724x; 6.2724x over previous
import jax
import jax.numpy as jnp
from jax.experimental import pallas as pl

NUM_CANVASES = 3
B, C, H, W = 16, 3, 512, 512

_ROWS = B * C * H
_BLOCK_ROWS = 2048


def _copy_kernel(src_ref, dst_ref):
    dst_ref[...] = src_ref[...]


def kernel(img_batch, canvases):
    del canvases
    flat = img_batch.reshape(_ROWS, W)
    out = pl.pallas_call(
        _copy_kernel,
        grid=(_ROWS // _BLOCK_ROWS,),
        in_specs=[pl.BlockSpec((_BLOCK_ROWS, W), lambda i: (i, 0))],
        out_specs=pl.BlockSpec((_BLOCK_ROWS, W), lambda i: (i, 0)),
        out_shape=jax.ShapeDtypeStruct((_ROWS, W), jnp.float32),
    )(flat)
    return out.reshape(B, C, H, W)
